# baseline (speedup 1.0000x reference)
import functools

import jax
import jax.numpy as jnp
from jax import lax
from jax.experimental import pallas as pl
from jax.experimental.pallas import tpu as pltpu
from jax.experimental.pallas import tpu_sc as plsc

_N = 10000
_NPAD = 10240
_E = 320000
_D = 128
_NC = 2
_NS = 16
_L = 16


@functools.cache
def _sc_mesh():
    return plsc.VectorSubcoreMesh(
        core_axis_name="c", subcore_axis_name="s", num_cores=_NC, num_subcores=_NS
    )


_DH = _D // _NC
_CH = 80
_NB = 4
_EW1 = _E // _NS
_CPW1 = _EW1 // _CH
_RPS = _NPAD // _NS


def _agg_rows_body(x2_hbm, z_hbm, src_hbm, dst_hbm, out_hbm,
                   srcall, dstall, srcc, dstc, rows, acc, gs, ss):
    c = lax.axis_index("c")
    s = lax.axis_index("s")
    pltpu.sync_copy(z_hbm.at[pl.ds(s * _RPS, _RPS)], acc.at[pl.ds(s * _RPS, _RPS)])
    pltpu.sync_copy(src_hbm.at[pl.ds(s * _EW1, _EW1)], srcall)
    pltpu.sync_copy(dst_hbm.at[pl.ds(s * _EW1, _EW1)], dstall)
    plsc.subcore_barrier()
    half = c * _N

    def start_gather(j, b, first):
        if not first:
            pltpu.make_async_copy(rows.at[b], acc.at[dstc.at[b]], ss.at[b]).wait()
        for k in range(_CH // _L):
            srcc[b, pl.ds(k * _L, _L)] = srcall[pl.ds(j * _CH + k * _L, _L)] + half
            dstc[b, pl.ds(k * _L, _L)] = dstall[pl.ds(j * _CH + k * _L, _L)]
        pltpu.async_copy(x2_hbm.at[srcc.at[b]], rows.at[b], gs.at[b])

    def start_scatter(b):
        pltpu.make_async_copy(x2_hbm.at[srcc.at[b]], rows.at[b], gs.at[b]).wait()
        pltpu.async_copy(rows.at[b], acc.at[dstc.at[b]], ss.at[b], add=True)

    start_gather(0, 0, True)
    start_gather(1, 1, True)
    start_scatter(0)
    start_gather(2, 2, True)
    start_scatter(1)
    start_gather(3, 3, True)
    start_scatter(2)

    def quad(q, carry):
        j = 4 * q
        start_gather(j, 0, False)
        start_scatter(3)
        start_gather(j + 1, 1, False)
        start_scatter(0)
        start_gather(j + 2, 2, False)
        start_scatter(1)
        start_gather(j + 3, 3, False)
        start_scatter(2)
        return carry

    lax.fori_loop(1, _CPW1 // 4 - 1, quad, 0)
    for j, b in ((244, 0), (245, 1), (246, 2), (247, 3), (248, 0), (249, 1)):
        start_gather(j, b, False)
        start_scatter((b - 1) % _NB)
    start_scatter(1)
    for b in range(_NB):
        pltpu.make_async_copy(rows.at[b], acc.at[dstc.at[b]], ss.at[b]).wait()

    plsc.subcore_barrier()
    pltpu.sync_copy(acc.at[pl.ds(s * _RPS, _RPS)], out_hbm.at[c, pl.ds(s * _RPS, _RPS)])


@functools.cache
def _agg_rows():
    return pl.kernel(
        _agg_rows_body,
        out_type=jax.ShapeDtypeStruct((_NC, _NPAD, _DH), jnp.float32),
        mesh=_sc_mesh(),
        compiler_params=pltpu.CompilerParams(use_tc_tiling_on_sc=False),
        scratch_types=[
            pltpu.VMEM((_EW1,), jnp.int32),
            pltpu.VMEM((_EW1,), jnp.int32),
            pltpu.VMEM((_NB, _CH), jnp.int32),
            pltpu.VMEM((_NB, _CH), jnp.int32),
            pltpu.VMEM((_NB, _CH, _DH), jnp.float32),
            pltpu.VMEM_SHARED((_NPAD, _DH), jnp.float32),
            pltpu.SemaphoreType.DMA((_NB,)),
            pltpu.SemaphoreType.DMA((_NB,)),
        ],
    )


_RB = 2048


def _mlp_body(p_ref, w1_ref, w2_ref, g_ref):
    a = jnp.concatenate([p_ref[0], p_ref[1]], axis=-1)
    h1 = jnp.maximum(
        jnp.dot(a, w1_ref[...], preferred_element_type=jnp.float32), 0.0
    )
    g_ref[...] = jnp.dot(h1, w2_ref[...], preferred_element_type=jnp.float32)


def _mlp(p, W1, W2):
    return pl.pallas_call(
        _mlp_body,
        grid=(_NPAD // _RB,),
        in_specs=[
            pl.BlockSpec((2, _RB, _DH), lambda i: (0, i, 0)),
            pl.BlockSpec((_D, _D), lambda i: (0, 0)),
            pl.BlockSpec((_D, 1), lambda i: (0, 0)),
        ],
        out_specs=pl.BlockSpec((_RB, 1), lambda i: (i, 0)),
        out_shape=jax.ShapeDtypeStruct((_NPAD, 1), jnp.float32),
    )(p, W1, W2)


_CPW2 = _E // (_NS * _CH)
_PPS = _NPAD // _NS


def _agg_scalar_body(g_hbm, z2_hbm, src_hbm, dst_hbm, out_hbm,
                     gtab, srcv, dstv, msgv, vbuf, acc2, ss0, ss1):
    c = lax.axis_index("c")
    s = lax.axis_index("s")

    @pl.when(c == 0)
    def _():
        pltpu.sync_copy(z2_hbm.at[pl.ds(s * _PPS, _PPS)], acc2.at[pl.ds(s * _PPS, _PPS)])
        pltpu.sync_copy(g_hbm, gtab)
        pltpu.sync_copy(src_hbm.at[s], srcv)
        pltpu.sync_copy(dst_hbm.at[s], dstv)
        plsc.subcore_barrier()

        def chunk(j, b, ssem, first):
            if not first:
                pltpu.make_async_copy(msgv.at[b], acc2.at[dstv.at[j]], ssem).wait()
            for k in range(_CH // _L):
                idx = srcv[j, pl.ds(k * _L, _L)]
                msgv[b, pl.ds(k * _L, _L)] = plsc.load_gather(gtab, [idx])
            pltpu.async_copy(msgv.at[b], acc2.at[dstv.at[j]], ssem, add=True)

        def pair(g, carry):
            chunk(2 * g, 0, ss0, False)
            chunk(2 * g + 1, 1, ss1, False)
            return carry

        chunk(0, 0, ss0, True)
        chunk(1, 1, ss1, True)
        lax.fori_loop(1, _CPW2 // 2, pair, 0)
        pltpu.make_async_copy(msgv.at[0], acc2.at[dstv.at[_CPW2 - 2]], ss0).wait()
        pltpu.make_async_copy(msgv.at[1], acc2.at[dstv.at[_CPW2 - 1]], ss1).wait()

        plsc.subcore_barrier()
        pltpu.sync_copy(acc2.at[pl.ds(s * _PPS, _PPS)], vbuf)

        def relu_step(j, carry):
            vbuf[pl.ds(j * _L, _L)] = jnp.maximum(vbuf[pl.ds(j * _L, _L)], 0.0)
            return carry

        lax.fori_loop(0, _PPS // _L, relu_step, 0)
        pltpu.sync_copy(vbuf, out_hbm.at[pl.ds(s * _PPS, _PPS)])


@functools.cache
def _agg_scalar():
    return pl.kernel(
        _agg_scalar_body,
        out_type=jax.ShapeDtypeStruct((_NPAD,), jnp.float32),
        mesh=_sc_mesh(),
        compiler_params=pltpu.CompilerParams(needs_layout_passes=False),
        scratch_types=[
            pltpu.VMEM((_NPAD,), jnp.float32),
            pltpu.VMEM((_CPW2, _CH), jnp.int32),
            pltpu.VMEM((_CPW2, _CH), jnp.int32),
            pltpu.VMEM((2, _CH), jnp.float32),
            pltpu.VMEM((_PPS,), jnp.float32),
            pltpu.VMEM_SHARED((_NPAD,), jnp.float32),
            pltpu.SemaphoreType.DMA,
            pltpu.SemaphoreType.DMA,
        ],
    )


@jax.jit
def kernel(x, edge_index, batch, W1, W2):
    del batch
    src = edge_index[0]
    dst = edge_index[1]
    x2 = jnp.concatenate([x[:, :_DH], x[:, _DH:]], axis=0)
    z = jnp.zeros((_NPAD, _DH), jnp.float32)
    p = _agg_rows()(x2, z, src, dst)
    g = _mlp(p, W1, W2)
    src2 = src.reshape(_NS, _CPW2, _CH)
    dst2 = dst.reshape(_NS, _CPW2, _CH)
    z2 = jnp.zeros((_NPAD,), jnp.float32)
    o = _agg_scalar()(g.reshape(_NPAD), z2, src2, dst2)
    return o[:_N].reshape(_N, 1)


# speedup vs baseline: 1.1060x; 1.1060x over previous
import functools

import jax
import jax.numpy as jnp
from jax import lax
from jax.experimental import pallas as pl
from jax.experimental.pallas import tpu as pltpu
from jax.experimental.pallas import tpu_sc as plsc

_N = 10000
_NPAD = 10240
_E = 320000
_D = 128
_NC = 2
_NS = 16
_L = 16


@functools.cache
def _sc_mesh():
    return plsc.VectorSubcoreMesh(
        core_axis_name="c", subcore_axis_name="s", num_cores=_NC, num_subcores=_NS
    )


_DH = _D // _NC
_CH = 80
_NB = 4
_EW1 = _E // _NS
_CPW1 = _EW1 // _CH
_RPS = _NPAD // _NS


def _agg_rows_body(x2_hbm, src_hbm, dst_hbm, out_hbm,
                   srcall, dstall, srcc, dstc, rows, acc, gs, ss):
    c = lax.axis_index("c")
    s = lax.axis_index("s")
    pltpu.sync_copy(src_hbm.at[pl.ds(s * _EW1, _EW1)], srcall)
    pltpu.sync_copy(dst_hbm.at[pl.ds(s * _EW1, _EW1)], dstall)
    def zero_step(i, carry):
        rows[0, i // (_DH // _L), pl.ds((i % (_DH // _L)) * _L, _L)] = (
            jnp.zeros((_L,), jnp.float32))
        return carry
    lax.fori_loop(0, _CH * _DH // _L, zero_step, 0)
    for t in range(_RPS // _CH):
        pltpu.sync_copy(rows.at[0], acc.at[pl.ds(s * _RPS + t * _CH, _CH)])
    plsc.subcore_barrier()

    def start_gather(j, b, first):
        if not first:
            pltpu.make_async_copy(rows.at[b], acc.at[dstc.at[b]], ss.at[b]).wait()
        for k in range(_CH // _L):
            srcc[b, pl.ds(k * _L, _L)] = (
                srcall[pl.ds(j * _CH + k * _L, _L)] * 2 + c)
            dstc[b, pl.ds(k * _L, _L)] = dstall[pl.ds(j * _CH + k * _L, _L)]
        pltpu.async_copy(x2_hbm.at[srcc.at[b]], rows.at[b], gs.at[b])

    def start_scatter(b):
        pltpu.make_async_copy(x2_hbm.at[srcc.at[b]], rows.at[b], gs.at[b]).wait()
        pltpu.async_copy(rows.at[b], acc.at[dstc.at[b]], ss.at[b], add=True)

    start_gather(0, 0, True)
    start_gather(1, 1, True)
    start_scatter(0)
    start_gather(2, 2, True)
    start_scatter(1)
    start_gather(3, 3, True)
    start_scatter(2)

    def quad(q, carry):
        j = 4 * q
        start_gather(j, 0, False)
        start_scatter(3)
        start_gather(j + 1, 1, False)
        start_scatter(0)
        start_gather(j + 2, 2, False)
        start_scatter(1)
        start_gather(j + 3, 3, False)
        start_scatter(2)
        return carry

    lax.fori_loop(1, _CPW1 // 4 - 1, quad, 0)
    for j, b in ((244, 0), (245, 1), (246, 2), (247, 3), (248, 0), (249, 1)):
        start_gather(j, b, False)
        start_scatter((b - 1) % _NB)
    start_scatter(1)
    for b in range(_NB):
        pltpu.make_async_copy(rows.at[b], acc.at[dstc.at[b]], ss.at[b]).wait()

    plsc.subcore_barrier()
    pltpu.sync_copy(acc.at[pl.ds(s * _RPS, _RPS)], out_hbm.at[c, pl.ds(s * _RPS, _RPS)])


@functools.cache
def _agg_rows():
    return pl.kernel(
        _agg_rows_body,
        out_type=jax.ShapeDtypeStruct((_NC, _NPAD, _DH), jnp.float32),
        mesh=_sc_mesh(),
        compiler_params=pltpu.CompilerParams(use_tc_tiling_on_sc=False),
        scratch_types=[
            pltpu.VMEM((_EW1,), jnp.int32),
            pltpu.VMEM((_EW1,), jnp.int32),
            pltpu.VMEM((_NB, _CH), jnp.int32),
            pltpu.VMEM((_NB, _CH), jnp.int32),
            pltpu.VMEM((_NB, _CH, _DH), jnp.float32),
            pltpu.VMEM_SHARED((_NPAD, _DH), jnp.float32),
            pltpu.SemaphoreType.DMA((_NB,)),
            pltpu.SemaphoreType.DMA((_NB,)),
        ],
    )


_RB = 2048


def _mlp_body(p_ref, w1_ref, w2_ref, g_ref):
    a = jnp.concatenate([p_ref[0], p_ref[1]], axis=-1)
    h1 = jnp.maximum(
        jnp.dot(a, w1_ref[...], preferred_element_type=jnp.float32), 0.0
    )
    g_ref[...] = jnp.dot(h1, w2_ref[...], preferred_element_type=jnp.float32)


def _mlp(p, W1, W2):
    return pl.pallas_call(
        _mlp_body,
        grid=(_NPAD // _RB,),
        in_specs=[
            pl.BlockSpec((2, _RB, _DH), lambda i: (0, i, 0)),
            pl.BlockSpec((_D, _D), lambda i: (0, 0)),
            pl.BlockSpec((_D, 1), lambda i: (0, 0)),
        ],
        out_specs=pl.BlockSpec((_RB, 1), lambda i: (i, 0)),
        out_shape=jax.ShapeDtypeStruct((_NPAD, 1), jnp.float32),
    )(p, W1, W2)


_CPW2 = _E // (_NS * _CH)
_PPS = _NPAD // _NS


def _agg_scalar_body(g_hbm, src_hbm, dst_hbm, out_hbm,
                     gtab, srcv, dstv, msgv, vbuf, acc2, ss0, ss1):
    c = lax.axis_index("c")
    s = lax.axis_index("s")

    @pl.when(c == 0)
    def _():
        def zero_step(i, carry):
            vbuf[pl.ds(i * _L, _L)] = jnp.zeros((_L,), jnp.float32)
            return carry
        lax.fori_loop(0, _PPS // _L, zero_step, 0)
        pltpu.sync_copy(vbuf, acc2.at[pl.ds(s * _PPS, _PPS)])
        pltpu.sync_copy(g_hbm, gtab)
        pltpu.sync_copy(src_hbm.at[s], srcv)
        pltpu.sync_copy(dst_hbm.at[s], dstv)
        plsc.subcore_barrier()

        def chunk(j, b, ssem, first):
            if not first:
                pltpu.make_async_copy(msgv.at[b], acc2.at[dstv.at[j]], ssem).wait()
            for k in range(_CH // _L):
                idx = srcv[j, pl.ds(k * _L, _L)]
                msgv[b, pl.ds(k * _L, _L)] = plsc.load_gather(gtab, [idx])
            pltpu.async_copy(msgv.at[b], acc2.at[dstv.at[j]], ssem, add=True)

        def pair(g, carry):
            chunk(2 * g, 0, ss0, False)
            chunk(2 * g + 1, 1, ss1, False)
            return carry

        chunk(0, 0, ss0, True)
        chunk(1, 1, ss1, True)
        lax.fori_loop(1, _CPW2 // 2, pair, 0)
        pltpu.make_async_copy(msgv.at[0], acc2.at[dstv.at[_CPW2 - 2]], ss0).wait()
        pltpu.make_async_copy(msgv.at[1], acc2.at[dstv.at[_CPW2 - 1]], ss1).wait()

        plsc.subcore_barrier()
        pltpu.sync_copy(acc2.at[pl.ds(s * _PPS, _PPS)], vbuf)

        def relu_step(j, carry):
            vbuf[pl.ds(j * _L, _L)] = jnp.maximum(vbuf[pl.ds(j * _L, _L)], 0.0)
            return carry

        lax.fori_loop(0, _PPS // _L, relu_step, 0)
        pltpu.sync_copy(vbuf, out_hbm.at[pl.ds(s * _PPS, _PPS)])


@functools.cache
def _agg_scalar():
    return pl.kernel(
        _agg_scalar_body,
        out_type=jax.ShapeDtypeStruct((_NPAD,), jnp.float32),
        mesh=_sc_mesh(),
        compiler_params=pltpu.CompilerParams(needs_layout_passes=False),
        scratch_types=[
            pltpu.VMEM((_NPAD,), jnp.float32),
            pltpu.VMEM((_CPW2, _CH), jnp.int32),
            pltpu.VMEM((_CPW2, _CH), jnp.int32),
            pltpu.VMEM((2, _CH), jnp.float32),
            pltpu.VMEM((_PPS,), jnp.float32),
            pltpu.VMEM_SHARED((_NPAD,), jnp.float32),
            pltpu.SemaphoreType.DMA,
            pltpu.SemaphoreType.DMA,
        ],
    )


@jax.jit
def kernel(x, edge_index, batch, W1, W2):
    del batch
    src = edge_index[0]
    dst = edge_index[1]
    x2 = x.reshape(2 * _N, _DH)
    p = _agg_rows()(x2, src, dst)
    g = _mlp(p, W1, W2)
    src2 = src.reshape(_NS, _CPW2, _CH)
    dst2 = dst.reshape(_NS, _CPW2, _CH)
    o = _agg_scalar()(g.reshape(_NPAD), src2, dst2)
    return o[:_N].reshape(_N, 1)


# speedup vs baseline: 1.1062x; 1.0002x over previous
import functools

import jax
import jax.numpy as jnp
from jax import lax
from jax.experimental import pallas as pl
from jax.experimental.pallas import tpu as pltpu
from jax.experimental.pallas import tpu_sc as plsc

_N = 10000
_NPAD = 10240
_E = 320000
_D = 128
_NC = 2
_NS = 16
_L = 16


@functools.cache
def _sc_mesh():
    return plsc.VectorSubcoreMesh(
        core_axis_name="c", subcore_axis_name="s", num_cores=_NC, num_subcores=_NS
    )


_DH = _D // _NC
_CH = 80
_NB = 6
_EW1 = _E // _NS
_CPW1 = _EW1 // _CH
_RPS = _NPAD // _NS


def _agg_rows_body(x2_hbm, src_hbm, dst_hbm, out_hbm,
                   srcall, dstall, srcc, dstc, rows, acc, gs, ss):
    c = lax.axis_index("c")
    s = lax.axis_index("s")
    pltpu.sync_copy(src_hbm.at[pl.ds(s * _EW1, _EW1)], srcall)
    pltpu.sync_copy(dst_hbm.at[pl.ds(s * _EW1, _EW1)], dstall)
    def zero_step(i, carry):
        rows[0, i // (_DH // _L), pl.ds((i % (_DH // _L)) * _L, _L)] = (
            jnp.zeros((_L,), jnp.float32))
        return carry
    lax.fori_loop(0, _CH * _DH // _L, zero_step, 0)
    for t in range(_RPS // _CH):
        pltpu.sync_copy(rows.at[0], acc.at[pl.ds(s * _RPS + t * _CH, _CH)])
    plsc.subcore_barrier()

    def start_gather(j, b, first):
        if not first:
            pltpu.make_async_copy(rows.at[b], acc.at[dstc.at[b]], ss.at[b]).wait()
        for k in range(_CH // _L):
            srcc[b, pl.ds(k * _L, _L)] = (
                srcall[pl.ds(j * _CH + k * _L, _L)] * 2 + c)
            dstc[b, pl.ds(k * _L, _L)] = dstall[pl.ds(j * _CH + k * _L, _L)]
        pltpu.async_copy(x2_hbm.at[srcc.at[b]], rows.at[b], gs.at[b])

    def start_scatter(b):
        pltpu.make_async_copy(x2_hbm.at[srcc.at[b]], rows.at[b], gs.at[b]).wait()
        pltpu.async_copy(rows.at[b], acc.at[dstc.at[b]], ss.at[b], add=True)

    start_gather(0, 0, True)
    start_gather(1, 1, True)
    start_scatter(0)
    start_gather(2, 2, True)
    start_scatter(1)
    start_gather(3, 3, True)
    start_scatter(2)
    start_gather(4, 4, True)
    start_scatter(3)
    start_gather(5, 5, True)
    start_scatter(4)

    def sext(q, carry):
        j = 6 * q
        for b in range(_NB):
            start_gather(j + b, b, False)
            start_scatter((b - 1) % _NB)
        return carry

    lax.fori_loop(1, 41, sext, 0)
    for j, b in ((246, 0), (247, 1), (248, 2), (249, 3)):
        start_gather(j, b, False)
        start_scatter((b - 1) % _NB)
    start_scatter(3)
    for b in range(_NB):
        pltpu.make_async_copy(rows.at[b], acc.at[dstc.at[b]], ss.at[b]).wait()

    plsc.subcore_barrier()
    pltpu.sync_copy(acc.at[pl.ds(s * _RPS, _RPS)], out_hbm.at[c, pl.ds(s * _RPS, _RPS)])


@functools.cache
def _agg_rows():
    return pl.kernel(
        _agg_rows_body,
        out_type=jax.ShapeDtypeStruct((_NC, _NPAD, _DH), jnp.float32),
        mesh=_sc_mesh(),
        compiler_params=pltpu.CompilerParams(use_tc_tiling_on_sc=False),
        scratch_types=[
            pltpu.VMEM((_EW1,), jnp.int32),
            pltpu.VMEM((_EW1,), jnp.int32),
            pltpu.VMEM((_NB, _CH), jnp.int32),
            pltpu.VMEM((_NB, _CH), jnp.int32),
            pltpu.VMEM((_NB, _CH, _DH), jnp.float32),
            pltpu.VMEM_SHARED((_NPAD, _DH), jnp.float32),
            pltpu.SemaphoreType.DMA((_NB,)),
            pltpu.SemaphoreType.DMA((_NB,)),
        ],
    )


_RB = 2048


def _mlp_body(p_ref, w1_ref, w2_ref, g_ref):
    a = jnp.concatenate([p_ref[0], p_ref[1]], axis=-1)
    h1 = jnp.maximum(
        jnp.dot(a, w1_ref[...], preferred_element_type=jnp.float32), 0.0
    )
    g_ref[...] = jnp.dot(h1, w2_ref[...], preferred_element_type=jnp.float32)


def _mlp(p, W1, W2):
    return pl.pallas_call(
        _mlp_body,
        grid=(_NPAD // _RB,),
        in_specs=[
            pl.BlockSpec((2, _RB, _DH), lambda i: (0, i, 0)),
            pl.BlockSpec((_D, _D), lambda i: (0, 0)),
            pl.BlockSpec((_D, 1), lambda i: (0, 0)),
        ],
        out_specs=pl.BlockSpec((_RB, 1), lambda i: (i, 0)),
        out_shape=jax.ShapeDtypeStruct((_NPAD, 1), jnp.float32),
    )(p, W1, W2)


_CPW2 = _E // (_NS * _CH)
_PPS = _NPAD // _NS


def _agg_scalar_body(g_hbm, src_hbm, dst_hbm, out_hbm,
                     gtab, srcv, dstv, msgv, vbuf, acc2, ss0, ss1):
    c = lax.axis_index("c")
    s = lax.axis_index("s")

    @pl.when(c == 0)
    def _():
        def zero_step(i, carry):
            vbuf[pl.ds(i * _L, _L)] = jnp.zeros((_L,), jnp.float32)
            return carry
        lax.fori_loop(0, _PPS // _L, zero_step, 0)
        pltpu.sync_copy(vbuf, acc2.at[pl.ds(s * _PPS, _PPS)])
        pltpu.sync_copy(g_hbm, gtab)
        pltpu.sync_copy(src_hbm.at[s], srcv)
        pltpu.sync_copy(dst_hbm.at[s], dstv)
        plsc.subcore_barrier()

        def chunk(j, b, ssem, first):
            if not first:
                pltpu.make_async_copy(msgv.at[b], acc2.at[dstv.at[j]], ssem).wait()
            for k in range(_CH // _L):
                idx = srcv[j, pl.ds(k * _L, _L)]
                msgv[b, pl.ds(k * _L, _L)] = plsc.load_gather(gtab, [idx])
            pltpu.async_copy(msgv.at[b], acc2.at[dstv.at[j]], ssem, add=True)

        def pair(g, carry):
            chunk(2 * g, 0, ss0, False)
            chunk(2 * g + 1, 1, ss1, False)
            return carry

        chunk(0, 0, ss0, True)
        chunk(1, 1, ss1, True)
        lax.fori_loop(1, _CPW2 // 2, pair, 0)
        pltpu.make_async_copy(msgv.at[0], acc2.at[dstv.at[_CPW2 - 2]], ss0).wait()
        pltpu.make_async_copy(msgv.at[1], acc2.at[dstv.at[_CPW2 - 1]], ss1).wait()

        plsc.subcore_barrier()
        pltpu.sync_copy(acc2.at[pl.ds(s * _PPS, _PPS)], vbuf)

        def relu_step(j, carry):
            vbuf[pl.ds(j * _L, _L)] = jnp.maximum(vbuf[pl.ds(j * _L, _L)], 0.0)
            return carry

        lax.fori_loop(0, _PPS // _L, relu_step, 0)
        pltpu.sync_copy(vbuf, out_hbm.at[pl.ds(s * _PPS, _PPS)])


@functools.cache
def _agg_scalar():
    return pl.kernel(
        _agg_scalar_body,
        out_type=jax.ShapeDtypeStruct((_NPAD,), jnp.float32),
        mesh=_sc_mesh(),
        compiler_params=pltpu.CompilerParams(needs_layout_passes=False),
        scratch_types=[
            pltpu.VMEM((_NPAD,), jnp.float32),
            pltpu.VMEM((_CPW2, _CH), jnp.int32),
            pltpu.VMEM((_CPW2, _CH), jnp.int32),
            pltpu.VMEM((2, _CH), jnp.float32),
            pltpu.VMEM((_PPS,), jnp.float32),
            pltpu.VMEM_SHARED((_NPAD,), jnp.float32),
            pltpu.SemaphoreType.DMA,
            pltpu.SemaphoreType.DMA,
        ],
    )


@jax.jit
def kernel(x, edge_index, batch, W1, W2):
    del batch
    src = edge_index[0]
    dst = edge_index[1]
    x2 = x.reshape(2 * _N, _DH)
    p = _agg_rows()(x2, src, dst)
    g = _mlp(p, W1, W2)
    src2 = src.reshape(_NS, _CPW2, _CH)
    dst2 = dst.reshape(_NS, _CPW2, _CH)
    o = _agg_scalar()(g.reshape(_NPAD), src2, dst2)
    return o[:_N].reshape(_N, 1)


# speedup vs baseline: 1.1307x; 1.0222x over previous
import functools

import jax
import jax.numpy as jnp
from jax import lax
from jax.experimental import pallas as pl
from jax.experimental.pallas import tpu as pltpu
from jax.experimental.pallas import tpu_sc as plsc

_N = 10000
_NPAD = 10240
_E = 320000
_D = 128
_NC = 2
_NS = 16
_L = 16


@functools.cache
def _sc_mesh():
    return plsc.VectorSubcoreMesh(
        core_axis_name="c", subcore_axis_name="s", num_cores=_NC, num_subcores=_NS
    )


_DH = _D // _NC
_CH = 80
_NB = 4
_EW1 = _E // _NS
_CPW1 = _EW1 // _CH
_RPS = _NPAD // _NS


def _agg_rows_body(x2_hbm, src_hbm, dst_hbm, out_hbm,
                   srcall, dstall, srcc, dstc, rows, acc, gs, ss):
    c = lax.axis_index("c")
    s = lax.axis_index("s")
    pltpu.sync_copy(src_hbm.at[pl.ds(s * _EW1, _EW1)], srcall)
    pltpu.sync_copy(dst_hbm.at[pl.ds(s * _EW1, _EW1)], dstall)
    def zero_step(i, carry):
        rows[0, i // (_DH // _L), pl.ds((i % (_DH // _L)) * _L, _L)] = (
            jnp.zeros((_L,), jnp.float32))
        return carry
    lax.fori_loop(0, _CH * _DH // _L, zero_step, 0)
    for t in range(_RPS // _CH):
        pltpu.sync_copy(rows.at[0], acc.at[pl.ds(s * _RPS + t * _CH, _CH)])
    plsc.subcore_barrier()

    def start_gather(j, b, first):
        if not first:
            pltpu.make_async_copy(rows.at[b], acc.at[dstc.at[b]], ss.at[b]).wait()
        for k in range(_CH // _L):
            srcc[b, pl.ds(k * _L, _L)] = (
                srcall[pl.ds(j * _CH + k * _L, _L)] * 2 + c)
            dstc[b, pl.ds(k * _L, _L)] = dstall[pl.ds(j * _CH + k * _L, _L)]
        pltpu.async_copy(x2_hbm.at[srcc.at[b]], rows.at[b], gs.at[b])

    def start_scatter(b):
        pltpu.make_async_copy(x2_hbm.at[srcc.at[b]], rows.at[b], gs.at[b]).wait()
        pltpu.async_copy(rows.at[b], acc.at[dstc.at[b]], ss.at[b], add=True)

    start_gather(0, 0, True)
    start_gather(1, 1, True)
    start_scatter(0)
    start_gather(2, 2, True)
    start_scatter(1)
    start_gather(3, 3, True)
    start_scatter(2)

    def quad(q, carry):
        j = 4 * q
        start_gather(j, 0, False)
        start_scatter(3)
        start_gather(j + 1, 1, False)
        start_scatter(0)
        start_gather(j + 2, 2, False)
        start_scatter(1)
        start_gather(j + 3, 3, False)
        start_scatter(2)
        return carry

    lax.fori_loop(1, _CPW1 // 4 - 1, quad, 0)
    for j, b in ((244, 0), (245, 1), (246, 2), (247, 3), (248, 0), (249, 1)):
        start_gather(j, b, False)
        start_scatter((b - 1) % _NB)
    start_scatter(1)
    for b in range(_NB):
        pltpu.make_async_copy(rows.at[b], acc.at[dstc.at[b]], ss.at[b]).wait()

    plsc.subcore_barrier()
    pltpu.sync_copy(acc.at[pl.ds(s * _RPS, _RPS)], out_hbm.at[c, pl.ds(s * _RPS, _RPS)])


@functools.cache
def _agg_rows():
    return pl.kernel(
        _agg_rows_body,
        out_type=jax.ShapeDtypeStruct((_NC, _NPAD, _DH), jnp.float32),
        mesh=_sc_mesh(),
        compiler_params=pltpu.CompilerParams(use_tc_tiling_on_sc=False),
        scratch_types=[
            pltpu.VMEM((_EW1,), jnp.int32),
            pltpu.VMEM((_EW1,), jnp.int32),
            pltpu.VMEM((_NB, _CH), jnp.int32),
            pltpu.VMEM((_NB, _CH), jnp.int32),
            pltpu.VMEM((_NB, _CH, _DH), jnp.float32),
            pltpu.VMEM_SHARED((_NPAD, _DH), jnp.float32),
            pltpu.SemaphoreType.DMA((_NB,)),
            pltpu.SemaphoreType.DMA((_NB,)),
        ],
    )


_RB = 2048


def _mlp_body(p_ref, w1_ref, w2_ref, g_ref):
    a = jnp.concatenate([p_ref[0], p_ref[1]], axis=-1)
    h1 = jnp.maximum(
        jnp.dot(a, w1_ref[...], preferred_element_type=jnp.float32), 0.0
    )
    g_ref[...] = jnp.dot(h1, w2_ref[...], preferred_element_type=jnp.float32)[:, 0]


def _mlp(p, W1, W2):
    return pl.pallas_call(
        _mlp_body,
        grid=(_NPAD // _RB,),
        in_specs=[
            pl.BlockSpec((2, _RB, _DH), lambda i: (0, i, 0)),
            pl.BlockSpec((_D, _D), lambda i: (0, 0)),
            pl.BlockSpec((_D, 1), lambda i: (0, 0)),
        ],
        out_specs=pl.BlockSpec((_RB,), lambda i: (i,)),
        out_shape=jax.ShapeDtypeStruct((_NPAD,), jnp.float32),
    )(p, W1, W2)


_CPW2 = _E // (_NS * _CH)
_PPS = _NPAD // _NS


def _agg_scalar_body(g_hbm, src_hbm, dst_hbm, out_hbm,
                     gtab, srcv, dstv, msgv, vbuf, acc2, ss0, ss1):
    c = lax.axis_index("c")
    s = lax.axis_index("s")

    @pl.when(c == 0)
    def _():
        def zero_step(i, carry):
            vbuf[pl.ds(i * _L, _L)] = jnp.zeros((_L,), jnp.float32)
            return carry
        lax.fori_loop(0, _PPS // _L, zero_step, 0)
        pltpu.sync_copy(vbuf, acc2.at[pl.ds(s * _PPS, _PPS)])
        pltpu.sync_copy(g_hbm, gtab)
        pltpu.sync_copy(src_hbm.at[s], srcv)
        pltpu.sync_copy(dst_hbm.at[s], dstv)
        plsc.subcore_barrier()

        def chunk(j, b, ssem, first):
            if not first:
                pltpu.make_async_copy(msgv.at[b], acc2.at[dstv.at[j]], ssem).wait()
            for k in range(_CH // _L):
                idx = srcv[j, pl.ds(k * _L, _L)]
                msgv[b, pl.ds(k * _L, _L)] = plsc.load_gather(gtab, [idx])
            pltpu.async_copy(msgv.at[b], acc2.at[dstv.at[j]], ssem, add=True)

        def pair(g, carry):
            chunk(2 * g, 0, ss0, False)
            chunk(2 * g + 1, 1, ss1, False)
            return carry

        chunk(0, 0, ss0, True)
        chunk(1, 1, ss1, True)
        lax.fori_loop(1, _CPW2 // 2, pair, 0)
        pltpu.make_async_copy(msgv.at[0], acc2.at[dstv.at[_CPW2 - 2]], ss0).wait()
        pltpu.make_async_copy(msgv.at[1], acc2.at[dstv.at[_CPW2 - 1]], ss1).wait()

        plsc.subcore_barrier()
        pltpu.sync_copy(acc2.at[pl.ds(s * _PPS, _PPS)], vbuf)

        def relu_step(j, carry):
            vbuf[pl.ds(j * _L, _L)] = jnp.maximum(vbuf[pl.ds(j * _L, _L)], 0.0)
            return carry

        lax.fori_loop(0, _PPS // _L, relu_step, 0)
        pltpu.sync_copy(vbuf, out_hbm.at[pl.ds(s * _PPS, _PPS)])


@functools.cache
def _agg_scalar():
    return pl.kernel(
        _agg_scalar_body,
        out_type=jax.ShapeDtypeStruct((_NPAD,), jnp.float32),
        mesh=_sc_mesh(),
        compiler_params=pltpu.CompilerParams(needs_layout_passes=False),
        scratch_types=[
            pltpu.VMEM((_NPAD,), jnp.float32),
            pltpu.VMEM((_CPW2, _CH), jnp.int32),
            pltpu.VMEM((_CPW2, _CH), jnp.int32),
            pltpu.VMEM((2, _CH), jnp.float32),
            pltpu.VMEM((_PPS,), jnp.float32),
            pltpu.VMEM_SHARED((_NPAD,), jnp.float32),
            pltpu.SemaphoreType.DMA,
            pltpu.SemaphoreType.DMA,
        ],
    )


@jax.jit
def kernel(x, edge_index, batch, W1, W2):
    del batch
    src = edge_index[0]
    dst = edge_index[1]
    x2 = x.reshape(2 * _N, _DH)
    p = _agg_rows()(x2, src, dst)
    g = _mlp(p, W1, W2)
    src2 = src.reshape(_NS, _CPW2, _CH)
    dst2 = dst.reshape(_NS, _CPW2, _CH)
    o = _agg_scalar()(g, src2, dst2)
    return o[:_N].reshape(_N, 1)
